# baseline (speedup 1.0000x reference)
import functools

import jax
import jax.numpy as jnp
from jax import lax
from jax.experimental import pallas as pl
from jax.experimental.pallas import tpu as pltpu
from jax.experimental.pallas import tpu_sc as plsc

NC = 2
NS = 16
NW = NC * NS
CHUNK = 128


def _sc_gather_body(per_w, emb, user_table, item_table, uids, iids,
                    out_u, out_i, uidx_v, iidx_v, sem):
    wid = lax.axis_index("s") * NC + lax.axis_index("c")
    base = wid * per_w
    pltpu.sync_copy(uids.at[pl.ds(base, per_w)], uidx_v)
    pltpu.sync_copy(iids.at[pl.ds(base, per_w)], iidx_v)

    def body(j, carry):
        k0 = j * 16
        uvec = uidx_v[pl.ds(k0, 16)]
        ivec = iidx_v[pl.ds(k0, 16)]
        for l in range(16):
            pltpu.async_copy(user_table.at[pl.ds(uvec[l], 1)],
                             out_u.at[pl.ds(base + k0 + l, 1)], sem)
            pltpu.async_copy(item_table.at[pl.ds(ivec[l], 1)],
                             out_i.at[pl.ds(base + k0 + l, 1)], sem)
        return carry

    lax.fori_loop(0, per_w // 16, body, 0)
    pltpu.make_async_copy(out_u.at[pl.ds(base, per_w)],
                          out_u.at[pl.ds(base, per_w)], sem).wait()
    pltpu.make_async_copy(out_i.at[pl.ds(base, per_w)],
                          out_i.at[pl.ds(base, per_w)], sem).wait()


@functools.partial(jax.jit, static_argnums=(0, 1))
def _sc_gather(per_w, emb, user_table, item_table, uids, iids):
    batch = per_w * NW
    mesh = plsc.VectorSubcoreMesh(core_axis_name="c", subcore_axis_name="s")
    fn = pl.kernel(
        functools.partial(_sc_gather_body, per_w, emb),
        out_type=(
            jax.ShapeDtypeStruct((batch, emb), jnp.float32),
            jax.ShapeDtypeStruct((batch, emb), jnp.float32),
        ),
        mesh=mesh,
        scratch_types=[
            pltpu.VMEM((per_w,), jnp.int32),
            pltpu.VMEM((per_w,), jnp.int32),
            pltpu.SemaphoreType.DMA,
        ],
    )
    return fn(user_table, item_table, uids, iids)


def _mlp_body(u_ref, v_ref, w1u_ref, w1i_ref, b1_ref, w2_ref, b2_ref,
              w3_ref, b3_ref, out_ref):
    dot = functools.partial(
        jnp.dot, preferred_element_type=jnp.float32,
        precision=lax.Precision.HIGHEST)
    h1 = (dot(u_ref[...], w1u_ref[...]) + dot(v_ref[...], w1i_ref[...])
          + b1_ref[...])
    h1 = jnp.where(h1 >= 0, h1, 0.01 * h1)
    h2 = dot(h1, w2_ref[...]) + b2_ref[...]
    h2 = jnp.where(h2 >= 0, h2, 0.01 * h2)
    out_ref[...] = dot(h2, w3_ref[...]) + b3_ref[...]


@functools.partial(jax.jit, static_argnums=(0,))
def _mlp(block_rows, u_emb, v_emb, w1u, w1i, b1, w2, b2, w3, b3):
    batch, emb = u_emb.shape
    d1 = w1u.shape[1]
    d2 = w2.shape[1]
    grid = (batch // block_rows,)
    full = lambda shape: pl.BlockSpec(shape, lambda i: (0, 0))
    return pl.pallas_call(
        _mlp_body,
        grid=grid,
        in_specs=[
            pl.BlockSpec((block_rows, emb), lambda i: (i, 0)),
            pl.BlockSpec((block_rows, emb), lambda i: (i, 0)),
            full((emb, d1)),
            full((emb, d1)),
            full((1, d1)),
            full((d1, d2)),
            full((1, d2)),
            full((d2, 1)),
            full((1, 1)),
        ],
        out_specs=pl.BlockSpec((block_rows, 1), lambda i: (i, 0)),
        out_shape=jax.ShapeDtypeStruct((batch, 1), jnp.float32),
    )(u_emb, v_emb, w1u, w1i, b1, w2, b2, w3, b3)


def kernel(user_ids, item_ids, user_table, item_table, W1, b1, W2, b2, W3, b3):
    batch = user_ids.shape[0]
    emb = user_table.shape[1]
    per_w = batch // NW
    u_emb, v_emb = _sc_gather(per_w, emb, user_table, item_table,
                              user_ids.astype(jnp.int32),
                              item_ids.astype(jnp.int32))
    w1t = W1.T
    w1u, w1i = w1t[:emb], w1t[emb:]
    y = _mlp(2048, u_emb, v_emb, w1u, w1i, b1.reshape(1, -1), W2.T,
             b2.reshape(1, -1), W3.T, b3.reshape(1, 1))
    return y.reshape(batch)


# speedup vs baseline: 1.1962x; 1.1962x over previous
import functools

import jax
import jax.numpy as jnp
from jax import lax
from jax.experimental import pallas as pl
from jax.experimental.pallas import tpu as pltpu
from jax.experimental.pallas import tpu_sc as plsc

NC = 2
NS = 16
NW = NC * NS


def _sc_gather_body(per_w, user_t, item_t, uids, iids, out_u, out_i,
                    uidx_v, iidx_v, urows_v, irows_v, usem, isem):
    wid = lax.axis_index("s") * NC + lax.axis_index("c")
    base = wid * per_w
    pltpu.sync_copy(uids.at[pl.ds(base, per_w)], uidx_v)
    pltpu.sync_copy(iids.at[pl.ds(base, per_w)], iidx_v)
    pltpu.async_copy(user_t.at[uidx_v], urows_v, usem)
    pltpu.async_copy(item_t.at[iidx_v], irows_v, isem)
    pltpu.make_async_copy(user_t.at[uidx_v], urows_v, usem).wait()
    pltpu.make_async_copy(item_t.at[iidx_v], irows_v, isem).wait()
    pltpu.sync_copy(urows_v, out_u.at[pl.ds(base, per_w)])
    pltpu.sync_copy(irows_v, out_i.at[pl.ds(base, per_w)])


@functools.partial(jax.jit, static_argnums=(0, 1))
def _sc_gather(per_w, emb, user_t, item_t, uids, iids):
    batch = per_w * NW
    mesh = plsc.VectorSubcoreMesh(core_axis_name="c", subcore_axis_name="s")
    fn = pl.kernel(
        functools.partial(_sc_gather_body, per_w),
        out_type=(
            jax.ShapeDtypeStruct((batch, emb), jnp.float32),
            jax.ShapeDtypeStruct((batch, emb), jnp.float32),
        ),
        mesh=mesh,
        compiler_params=pltpu.CompilerParams(use_tc_tiling_on_sc=False),
        scratch_types=[
            pltpu.VMEM((per_w,), jnp.int32),
            pltpu.VMEM((per_w,), jnp.int32),
            pltpu.VMEM((per_w, emb), jnp.float32),
            pltpu.VMEM((per_w, emb), jnp.float32),
            pltpu.SemaphoreType.DMA,
            pltpu.SemaphoreType.DMA,
        ],
    )
    return fn(user_t, item_t, uids, iids)


def _mlp_body(u_ref, v_ref, w1u_ref, w1i_ref, b1_ref, w2_ref, b2_ref,
              w3_ref, b3_ref, out_ref):
    dot = functools.partial(
        jnp.dot, preferred_element_type=jnp.float32,
        precision=lax.Precision.HIGHEST)
    h1 = (dot(u_ref[...], w1u_ref[...]) + dot(v_ref[...], w1i_ref[...])
          + b1_ref[...])
    h1 = jnp.where(h1 >= 0, h1, 0.01 * h1)
    h2 = dot(h1, w2_ref[...]) + b2_ref[...]
    h2 = jnp.where(h2 >= 0, h2, 0.01 * h2)
    out_ref[...] = dot(h2, w3_ref[...]) + b3_ref[...]


@functools.partial(jax.jit, static_argnums=(0,))
def _mlp(block_rows, u_emb, v_emb, w1u, w1i, b1, w2, b2, w3, b3):
    batch, emb = u_emb.shape
    d1 = w1u.shape[1]
    d2 = w2.shape[1]
    grid = (batch // block_rows,)
    full = lambda shape: pl.BlockSpec(shape, lambda i: (0, 0))
    return pl.pallas_call(
        _mlp_body,
        grid=grid,
        in_specs=[
            pl.BlockSpec((block_rows, emb), lambda i: (i, 0)),
            pl.BlockSpec((block_rows, emb), lambda i: (i, 0)),
            full((emb, d1)),
            full((emb, d1)),
            full((1, d1)),
            full((d1, d2)),
            full((1, d2)),
            full((d2, 1)),
            full((1, 1)),
        ],
        out_specs=pl.BlockSpec((block_rows, 1), lambda i: (i, 0)),
        out_shape=jax.ShapeDtypeStruct((batch, 1), jnp.float32),
    )(u_emb, v_emb, w1u, w1i, b1, w2, b2, w3, b3)


def kernel(user_ids, item_ids, user_table, item_table, W1, b1, W2, b2, W3, b3):
    batch = user_ids.shape[0]
    emb = user_table.shape[1]
    per_w = batch // NW
    u_emb, v_emb = _sc_gather(per_w, emb, user_table, item_table,
                              user_ids.astype(jnp.int32),
                              item_ids.astype(jnp.int32))
    w1t = W1.T
    w1u, w1i = w1t[:emb], w1t[emb:]
    y = _mlp(2048, u_emb, v_emb, w1u, w1i, b1.reshape(1, -1), W2.T,
             b2.reshape(1, -1), W3.T, b3.reshape(1, 1))
    return y.reshape(batch)


# speedup vs baseline: 2.3114x; 1.9324x over previous
import functools

import jax
import jax.numpy as jnp
from jax import lax
from jax.experimental import pallas as pl
from jax.experimental.pallas import tpu as pltpu
from jax.experimental.pallas import tpu_sc as plsc

NC = 2
NS = 16
NW = NC * NS
CHUNK = 32


def _sc_gather_body(per_w, emb, user_t, item_t, uids, iids, out_u, out_i,
                    uidx_v, iidx_v, gbuf_u, gbuf_i,
                    uout_v, iout_v, usem, isem):
    wid = lax.axis_index("s") * NC + lax.axis_index("c")
    base = wid * per_w
    pltpu.sync_copy(uids.at[pl.ds(base, per_w)], uidx_v)
    pltpu.sync_copy(iids.at[pl.ds(base, per_w)], iidx_v)
    iota = lax.iota(jnp.int32, 16)

    def chunk(c, carry):
        off = c * CHUNK
        rems = []
        gids = []
        for g in range(CHUNK // 16):
            vu = uidx_v[pl.ds(off + g * 16, 16)]
            vi = iidx_v[pl.ds(off + g * 16, 16)]
            rems.append((vu & 7, vi & 7))
            gids.append((lax.shift_right_logical(vu, 3),
                         lax.shift_right_logical(vi, 3)))
        for g in range(CHUNK // 16):
            gu_v, gi_v = gids[g]
            for j in range(16):
                k = g * 16 + j
                pltpu.async_copy(user_t.at[pl.ds(gu_v[j], 1)],
                                 gbuf_u.at[pl.ds(k, 1)], usem)
                pltpu.async_copy(item_t.at[pl.ds(gi_v[j], 1)],
                                 gbuf_i.at[pl.ds(k, 1)], isem)
        for k in range(CHUNK):
            pltpu.make_async_copy(user_t.at[pl.ds(0, 1)],
                                  gbuf_u.at[pl.ds(k, 1)], usem).wait()
            pltpu.make_async_copy(item_t.at[pl.ds(0, 1)],
                                  gbuf_i.at[pl.ds(k, 1)], isem).wait()
        for g in range(CHUNK // 16):
            l = iota + (g * 16)
            ru, ri = rems[g]
            for col in range(emb):
                fb = (off + g * 16) * emb + col + iota * emb
                cvec = jnp.full((16,), col, jnp.int32)
                uvals = plsc.load_gather(gbuf_u, [l, ru, cvec])
                ivals = plsc.load_gather(gbuf_i, [l, ri, cvec])
                plsc.store_scatter(uout_v, [fb], uvals)
                plsc.store_scatter(iout_v, [fb], ivals)
        return carry

    lax.fori_loop(0, per_w // CHUNK, chunk, 0)
    pltpu.sync_copy(uout_v, out_u.at[pl.ds(base * emb, per_w * emb)])
    pltpu.sync_copy(iout_v, out_i.at[pl.ds(base * emb, per_w * emb)])


@functools.partial(jax.jit, static_argnums=(0, 1))
def _sc_gather(per_w, emb, user_t, item_t, uids, iids):
    batch = per_w * NW
    mesh = plsc.VectorSubcoreMesh(core_axis_name="c", subcore_axis_name="s")
    fn = pl.kernel(
        functools.partial(_sc_gather_body, per_w, emb),
        out_type=(
            jax.ShapeDtypeStruct((batch * emb,), jnp.float32),
            jax.ShapeDtypeStruct((batch * emb,), jnp.float32),
        ),
        mesh=mesh,
        compiler_params=pltpu.CompilerParams(needs_layout_passes=False),
        scratch_types=[
            pltpu.VMEM((per_w,), jnp.int32),
            pltpu.VMEM((per_w,), jnp.int32),
            pltpu.VMEM((CHUNK, 8, emb), jnp.float32),
            pltpu.VMEM((CHUNK, 8, emb), jnp.float32),
            pltpu.VMEM((per_w * emb,), jnp.float32),
            pltpu.VMEM((per_w * emb,), jnp.float32),
            pltpu.SemaphoreType.DMA,
            pltpu.SemaphoreType.DMA,
        ],
    )
    return fn(user_t, item_t, uids, iids)


def _mlp_body(u_ref, v_ref, w1u_ref, w1i_ref, b1_ref, w2_ref, b2_ref,
              w3_ref, b3_ref, out_ref):
    dot = functools.partial(
        jnp.dot, preferred_element_type=jnp.float32,
        precision=lax.Precision.HIGHEST)
    h1 = (dot(u_ref[...], w1u_ref[...]) + dot(v_ref[...], w1i_ref[...])
          + b1_ref[...])
    h1 = jnp.where(h1 >= 0, h1, 0.01 * h1)
    h2 = dot(h1, w2_ref[...]) + b2_ref[...]
    h2 = jnp.where(h2 >= 0, h2, 0.01 * h2)
    out_ref[...] = dot(h2, w3_ref[...]) + b3_ref[...]


@functools.partial(jax.jit, static_argnums=(0,))
def _mlp(block_rows, u_emb, v_emb, w1u, w1i, b1, w2, b2, w3, b3):
    batch, emb = u_emb.shape
    d1 = w1u.shape[1]
    d2 = w2.shape[1]
    grid = (batch // block_rows,)
    full = lambda shape: pl.BlockSpec(shape, lambda i: (0, 0))
    return pl.pallas_call(
        _mlp_body,
        grid=grid,
        in_specs=[
            pl.BlockSpec((block_rows, emb), lambda i: (i, 0)),
            pl.BlockSpec((block_rows, emb), lambda i: (i, 0)),
            full((emb, d1)),
            full((emb, d1)),
            full((1, d1)),
            full((d1, d2)),
            full((1, d2)),
            full((d2, 1)),
            full((1, 1)),
        ],
        out_specs=pl.BlockSpec((block_rows, 1), lambda i: (i, 0)),
        out_shape=jax.ShapeDtypeStruct((batch, 1), jnp.float32),
    )(u_emb, v_emb, w1u, w1i, b1, w2, b2, w3, b3)


def kernel(user_ids, item_ids, user_table, item_table, W1, b1, W2, b2, W3, b3):
    batch = user_ids.shape[0]
    nrows, emb = user_table.shape
    per_w = batch // NW
    u_flat, v_flat = _sc_gather(
        per_w, emb,
        user_table.reshape(nrows // 8, 8, emb),
        item_table.reshape(nrows // 8, 8, emb),
        user_ids.astype(jnp.int32), item_ids.astype(jnp.int32))
    u_emb = u_flat.reshape(batch, emb)
    v_emb = v_flat.reshape(batch, emb)
    w1t = W1.T
    w1u, w1i = w1t[:emb], w1t[emb:]
    y = _mlp(2048, u_emb, v_emb, w1u, w1i, b1.reshape(1, -1), W2.T,
             b2.reshape(1, -1), W3.T, b3.reshape(1, 1))
    return y.reshape(batch)
